# baseline (speedup 1.0000x reference)
import functools

import jax
import jax.numpy as jnp
from jax.experimental import pallas as pl
from jax.experimental.pallas import tpu as pltpu

N = 10000
B = 32
LATENT = 64
FS2 = 64
NBLK = 10
TCOL = N * FS2 // NBLK


def _elu(v):
    return jnp.where(v > 0, v, jnp.expm1(v))


def _decode_body(z_ref, wd_ref, bd_ref, out_ref):
    y = jnp.dot(z_ref[...], wd_ref[...], preferred_element_type=jnp.float32)
    y = y + bd_ref[...]
    out_ref[...] = jnp.where(y > 0, y, jnp.exp(jnp.minimum(y, 0.0)) - 1.0)


def _decode(z, Wd, bd):
    return pl.pallas_call(
        _decode_body,
        grid=(NBLK,),
        in_specs=[
            pl.BlockSpec((B, LATENT), lambda i: (0, 0)),
            pl.BlockSpec((LATENT, TCOL), lambda i: (0, i)),
            pl.BlockSpec((1, TCOL), lambda i: (0, i)),
        ],
        out_specs=pl.BlockSpec((B, TCOL), lambda i: (0, i)),
        out_shape=jax.ShapeDtypeStruct((B, N * FS2), jnp.float32),
    )(z, Wd, bd.reshape(1, -1))


def _lmul(t, src, dst, norm):
    msg = t[:, src, :] * norm[None, :, None]
    return jnp.zeros_like(t).at[:, dst, :].add(msg)


def _cheb(t, src, dst, norm, W, b):
    Tx0 = t
    out = jnp.einsum('bnc,co->bno', Tx0, W[0])
    Tx1 = _lmul(Tx0, src, dst, norm)
    out = out + jnp.einsum('bnc,co->bno', Tx1, W[1])
    for k in range(2, W.shape[0]):
        Tx2 = 2.0 * _lmul(Tx1, src, dst, norm) - Tx0
        out = out + jnp.einsum('bnc,co->bno', Tx2, W[k])
        Tx0, Tx1 = Tx1, Tx2
    return out + b


def kernel(x, W1, b1, W2, b2, W3, b3, W4, b4, Wd, bd, Wc1, bc1, Wc2, bc2, Wc3, bc3, edge_index):
    src = edge_index[0]
    dst = edge_index[1]
    deg = jnp.maximum(jnp.bincount(dst, length=N), 1).astype(jnp.float32)
    norm = -1.0 / jnp.sqrt(deg[src] * deg[dst])
    h = _elu(x @ W1 + b1)
    h = _elu(h @ W2 + b2)
    h = _elu(h @ W3 + b3)
    z = h @ W4 + b4
    y = _decode(z, Wd, bd).reshape(B, N, FS2)
    y = _elu(_cheb(y, src, dst, norm, Wc1, bc1))
    y = _elu(_cheb(y, src, dst, norm, Wc2, bc2))
    y = _cheb(y, src, dst, norm, Wc3, bc3)
    return y


# speedup vs baseline: 11.7370x; 11.7370x over previous
import functools

import jax
import jax.numpy as jnp
from jax import lax
from jax.experimental import pallas as pl
from jax.experimental.pallas import tpu as pltpu
from jax.experimental.pallas import tpu_sc as plsc

N = 10000
E = 160000
B = 32
LATENT = 64
FS2 = 64
NBLK = 10
TCOL = N * FS2 // NBLK

NTILE = 16
KCH = 80
EPAD = KCH * 128 * NTILE
NP = 10240


def _elu(v):
    return jnp.where(v > 0, v, jnp.expm1(v))


def _decode_body(z_ref, wd_ref, bd_ref, out_ref):
    y = jnp.dot(z_ref[...], wd_ref[...], preferred_element_type=jnp.float32)
    y = y + bd_ref[...]
    out_ref[...] = jnp.where(y > 0, y, jnp.exp(jnp.minimum(y, 0.0)) - 1.0)


def _decode(z, Wd, bd):
    return pl.pallas_call(
        _decode_body,
        grid=(NBLK,),
        in_specs=[
            pl.BlockSpec((B, LATENT), lambda i: (0, 0)),
            pl.BlockSpec((LATENT, TCOL), lambda i: (0, i)),
            pl.BlockSpec((1, TCOL), lambda i: (0, i)),
        ],
        out_specs=pl.BlockSpec((B, TCOL), lambda i: (0, i)),
        out_shape=jax.ShapeDtypeStruct((B, N * FS2), jnp.float32),
    )(z, Wd, bd.reshape(1, -1))


def _make_gs(P):
    BP = P // 2
    mesh = plsc.VectorSubcoreMesh(core_axis_name="c", subcore_axis_name="s")

    @functools.partial(
        pl.kernel,
        mesh=mesh,
        out_type=jax.ShapeDtypeStruct((P * NP, 128), jnp.float32),
        scratch_types=[
            pltpu.VMEM((KCH, 128), jnp.int32),
            pltpu.VMEM((KCH, 128), jnp.int32),
            pltpu.VMEM((128, 128), jnp.float32),
            pltpu.VMEM_SHARED((NP, 128), jnp.float32),
            pltpu.SemaphoreType.DMA,
        ],
    )
    def gs(xf, idxs, dst2d, zrows, out, idx_v, dst_v, rows, acc, gsem):
        cc = lax.axis_index("c")
        ss = lax.axis_index("s")
        pltpu.sync_copy(dst2d.at[pl.ds(ss * KCH, KCH)], dst_v)

        def bbody(bl, carry):
            bg = cc * BP + bl
            pltpu.sync_copy(zrows, acc.at[pl.ds(ss * 640, 640)])
            pltpu.sync_copy(idxs.at[pl.ds(bg * (NTILE * KCH) + ss * KCH, KCH)], idx_v)
            plsc.subcore_barrier()

            def cbody(j, c2):
                pltpu.async_copy(xf.at[idx_v.at[j]], rows, gsem).wait()
                pltpu.sync_copy(rows, acc.at[dst_v.at[j]], add=True)
                return c2

            lax.fori_loop(0, KCH, cbody, 0)
            plsc.subcore_barrier()
            pltpu.sync_copy(acc.at[pl.ds(ss * 640, 640)],
                            out.at[pl.ds(bg * NP + ss * 640, 640)])
            plsc.subcore_barrier()
            return carry

        lax.fori_loop(0, BP, bbody, 0)

    return gs


_GS = {16: _make_gs(16), 8: _make_gs(8), 4: _make_gs(4)}


def _cheb_sc(t, src_p, dst2d, Wc, bc, sroot, invd):
    C = t.shape[2]
    q = 128 // C
    P = B // q
    gs = _GS[P]
    idxs = (src_p[None, :] + (jnp.arange(P, dtype=jnp.int32) * N)[:, None]
            ).reshape(P * NTILE * KCH, 128)

    def apply_g(vp):
        o = gs(vp.reshape(P * N, 128), idxs, dst2d,
               jnp.zeros((640, 128), jnp.float32))
        return o.reshape(P, NP, 128)[:, :N, :]

    yp = t * sroot[None, :, None]
    ypp = yp.reshape(P, q, N, C).transpose(0, 2, 1, 3).reshape(P, N, 128)
    a1 = -apply_g(ypp * invd[None, :, None])
    a2 = -2.0 * apply_g(a1 * invd[None, :, None]) - ypp
    O = Wc.shape[2]
    out = (jnp.einsum('pnqc,co->pnqo', ypp.reshape(P, N, q, C), Wc[0])
           + jnp.einsum('pnqc,co->pnqo', a1.reshape(P, N, q, C), Wc[1])
           + jnp.einsum('pnqc,co->pnqo', a2.reshape(P, N, q, C), Wc[2]))
    out = out * (1.0 / sroot)[None, :, None, None] + bc
    return out.transpose(0, 2, 1, 3).reshape(B, N, O)


def kernel(x, W1, b1, W2, b2, W3, b3, W4, b4, Wd, bd, Wc1, bc1, Wc2, bc2, Wc3, bc3, edge_index):
    src = edge_index[0]
    dst = edge_index[1]
    deg = jnp.maximum(jnp.bincount(dst, length=N), 1).astype(jnp.float32)
    sroot = jnp.sqrt(deg)
    invd = 1.0 / deg

    src_p = jnp.concatenate([src, jnp.zeros((EPAD - E,), jnp.int32)])
    dst_p = jnp.concatenate([dst, jnp.full((EPAD - E,), N, jnp.int32)])
    dst2d = dst_p.reshape(NTILE * KCH, 128)

    h = _elu(x @ W1 + b1)
    h = _elu(h @ W2 + b2)
    h = _elu(h @ W3 + b3)
    z = h @ W4 + b4
    y = _decode(z, Wd, bd).reshape(B, N, FS2)
    y = _elu(_cheb_sc(y, src_p, dst2d, Wc1, bc1, sroot, invd))
    y = _elu(_cheb_sc(y, src_p, dst2d, Wc2, bc2, sroot, invd))
    y = _cheb_sc(y, src_p, dst2d, Wc3, bc3, sroot, invd)
    return y


# speedup vs baseline: 14.0180x; 1.1943x over previous
import functools

import jax
import jax.numpy as jnp
from jax import lax
from jax.experimental import pallas as pl
from jax.experimental.pallas import tpu as pltpu
from jax.experimental.pallas import tpu_sc as plsc

N = 10000
E = 160000
B = 32
LATENT = 64
FS2 = 64
NBLK = 10
TCOL = N * FS2 // NBLK

NTILE = 16
KCH = 80
HCH = 40
EPAD = KCH * 128 * NTILE
NP = 10112


def _elu(v):
    return jnp.where(v > 0, v, jnp.expm1(v))


def _decode_body(z_ref, wd_ref, bd_ref, out_ref):
    y = jnp.dot(z_ref[...], wd_ref[...], preferred_element_type=jnp.float32)
    y = y + bd_ref[...]
    out_ref[...] = jnp.where(y > 0, y, jnp.exp(jnp.minimum(y, 0.0)) - 1.0)


def _decode(z, Wd, bd):
    return pl.pallas_call(
        _decode_body,
        grid=(NBLK,),
        in_specs=[
            pl.BlockSpec((B, LATENT), lambda i: (0, 0)),
            pl.BlockSpec((LATENT, TCOL), lambda i: (0, i)),
            pl.BlockSpec((1, TCOL), lambda i: (0, i)),
        ],
        out_specs=pl.BlockSpec((B, TCOL), lambda i: (0, i)),
        out_shape=jax.ShapeDtypeStruct((B, N * FS2), jnp.float32),
    )(z, Wd, bd.reshape(1, -1))


def _make_gs(P):
    BP = P // 2
    mesh = plsc.VectorSubcoreMesh(core_axis_name="c", subcore_axis_name="s")

    @functools.partial(
        pl.kernel,
        mesh=mesh,
        out_type=jax.ShapeDtypeStruct((P * NP, 128), jnp.float32),
        scratch_types=[
            pltpu.VMEM((HCH, 128), jnp.int32),
            pltpu.VMEM((HCH, 128), jnp.int32),
            pltpu.VMEM((128, 128), jnp.float32),
            pltpu.VMEM((128, 128), jnp.float32),
            pltpu.VMEM_SHARED((NP, 128), jnp.float32),
            pltpu.SemaphoreType.DMA,
            pltpu.SemaphoreType.DMA,
        ],
    )
    def gs(xf, idxs, dst2d, zrows, out, idx_v, dst_v, ra, rb, acc, sema, semb):
        cc = lax.axis_index("c")
        ss = lax.axis_index("s")

        def bbody(bl, carry):
            bg = cc * BP + bl
            pltpu.sync_copy(zrows, acc.at[pl.ds(ss * 632, 632)])
            plsc.subcore_barrier()

            def hbody(hh, c1):
                base = ss * KCH + hh * HCH
                pltpu.sync_copy(idxs.at[pl.ds(bg * (NTILE * KCH) + base, HCH)],
                                idx_v)
                pltpu.sync_copy(dst2d.at[pl.ds(base, HCH)], dst_v)
                pltpu.async_copy(xf.at[idx_v.at[0]], ra, sema)

                def cbody(t, c2):
                    j0 = 2 * t
                    pltpu.async_copy(xf.at[idx_v.at[j0 + 1]], rb, semb)
                    pltpu.make_async_copy(xf.at[idx_v.at[j0]], ra, sema).wait()
                    pltpu.sync_copy(ra, acc.at[dst_v.at[j0]], add=True)

                    @pl.when(t < HCH // 2 - 1)
                    def _refill():
                        pltpu.async_copy(xf.at[idx_v.at[j0 + 2]], ra, sema)

                    pltpu.make_async_copy(xf.at[idx_v.at[j0 + 1]], rb, semb).wait()
                    pltpu.sync_copy(rb, acc.at[dst_v.at[j0 + 1]], add=True)
                    return c2

                lax.fori_loop(0, HCH // 2, cbody, 0)
                return c1

            lax.fori_loop(0, 2, hbody, 0)
            plsc.subcore_barrier()
            pltpu.sync_copy(acc.at[pl.ds(ss * 632, 632)],
                            out.at[pl.ds(bg * NP + ss * 632, 632)])
            plsc.subcore_barrier()
            return carry

        lax.fori_loop(0, BP, bbody, 0)

    return gs


_GS = {16: _make_gs(16), 8: _make_gs(8), 4: _make_gs(4)}


def _cheb_sc(t, src_p, dst2d, Wc, bc, sroot, invd):
    C = t.shape[2]
    q = 128 // C
    P = B // q
    gs = _GS[P]
    idxs = (src_p[None, :] + (jnp.arange(P, dtype=jnp.int32) * N)[:, None]
            ).reshape(P * NTILE * KCH, 128)

    def apply_g(vp):
        o = gs(vp.reshape(P * N, 128), idxs, dst2d,
               jnp.zeros((632, 128), jnp.float32))
        return o.reshape(P, NP, 128)[:, :N, :]

    yp = t * sroot[None, :, None]
    ypp = yp.reshape(P, q, N, C).transpose(0, 2, 1, 3).reshape(P, N, 128)
    a1 = -apply_g(ypp * invd[None, :, None])
    a2 = -2.0 * apply_g(a1 * invd[None, :, None]) - ypp
    O = Wc.shape[2]
    out = (jnp.einsum('pnqc,co->pnqo', ypp.reshape(P, N, q, C), Wc[0])
           + jnp.einsum('pnqc,co->pnqo', a1.reshape(P, N, q, C), Wc[1])
           + jnp.einsum('pnqc,co->pnqo', a2.reshape(P, N, q, C), Wc[2]))
    out = out * (1.0 / sroot)[None, :, None, None] + bc
    return out.transpose(0, 2, 1, 3).reshape(B, N, O)


def kernel(x, W1, b1, W2, b2, W3, b3, W4, b4, Wd, bd, Wc1, bc1, Wc2, bc2, Wc3, bc3, edge_index):
    src = edge_index[0]
    dst = edge_index[1]
    deg = jnp.maximum(jnp.bincount(dst, length=N), 1).astype(jnp.float32)
    sroot = jnp.sqrt(deg)
    invd = 1.0 / deg

    src_p = jnp.concatenate([src, jnp.zeros((EPAD - E,), jnp.int32)])
    dst_p = jnp.concatenate([dst, jnp.full((EPAD - E,), N, jnp.int32)])
    dst2d = dst_p.reshape(NTILE * KCH, 128)

    h = _elu(x @ W1 + b1)
    h = _elu(h @ W2 + b2)
    h = _elu(h @ W3 + b3)
    z = h @ W4 + b4
    y = _decode(z, Wd, bd).reshape(B, N, FS2)
    y = _elu(_cheb_sc(y, src_p, dst2d, Wc1, bc1, sroot, invd))
    y = _elu(_cheb_sc(y, src_p, dst2d, Wc2, bc2, sroot, invd))
    y = _cheb_sc(y, src_p, dst2d, Wc3, bc3, sroot, invd)
    return y
